# Initial kernel scaffold; baseline (speedup 1.0000x reference)
import jax
import jax.numpy as jnp
from jax.experimental import pallas as pl


def kernel(scores, gating, segment_ids):
    raise NotImplementedError("write your pallas kernel here")


# speedup vs baseline: 1.8480x; 1.8480x over previous
import functools

import jax
import jax.numpy as jnp
from jax import lax
from jax.experimental import pallas as pl
from jax.experimental.pallas import tpu as pltpu
from jax.experimental.pallas import tpu_sc as plsc

_B = 16
_N = 32768
_NC = 2
_NS = 16
_L = 16
_NW = _NC * _NS
_C = _N // _NW
_V = _C // _L

_mesh = plsc.VectorSubcoreMesh(core_axis_name="c", subcore_axis_name="s")


@functools.partial(
    pl.kernel,
    mesh=_mesh,
    out_type=jax.ShapeDtypeStruct((_NW, _B), jnp.float32),
    scratch_types=[
        pltpu.VMEM((_C,), jnp.float32),
        pltpu.VMEM((_C,), jnp.float32),
        pltpu.VMEM((_C,), jnp.int32),
        pltpu.VMEM((_B,), jnp.float32),
        pltpu.VMEM((_NS, _B), jnp.float32),
        pltpu.VMEM_SHARED((_NS, _B), jnp.float32),
    ],
)
def _segsum_sc(scores_hbm, gating_hbm, ids_hbm, out_hbm,
               s_v, g_v, i_v, part_v, all_v, acc_sh):
    cid = lax.axis_index("c")
    sid = lax.axis_index("s")
    wid = sid * _NC + cid
    base = wid * _C

    pltpu.sync_copy(scores_hbm.at[pl.ds(base, _C)], s_v)
    pltpu.sync_copy(gating_hbm.at[pl.ds(base, _C)], g_v)
    pltpu.sync_copy(ids_hbm.at[pl.ds(base, _C)], i_v)

    def body(j, accs):
        sl = pl.ds(j * _L, _L)
        p = s_v[sl] * g_v[sl]
        seg = i_v[sl]
        return tuple(a + jnp.where(seg == b, p, 0.0)
                     for b, a in enumerate(accs))

    accs = lax.fori_loop(
        0, _V, body, tuple(jnp.zeros((_L,), jnp.float32) for _ in range(_B)))

    lanes = lax.iota(jnp.int32, _L)
    part = jnp.zeros((_L,), jnp.float32)
    for b in range(_B):
        tot = accs[b][0]
        for l in range(1, _L):
            tot = tot + accs[b][l]
        part = jnp.where(lanes == b, tot, part)
    part_v[...] = part

    pltpu.sync_copy(part_v, out_hbm.at[wid])


def kernel(scores, gating, segment_ids):
    partials = _segsum_sc(scores, gating, segment_ids.astype(jnp.int32))
    return jnp.sum(partials, axis=0)


# speedup vs baseline: 2.0077x; 1.0864x over previous
import functools

import jax
import jax.numpy as jnp
from jax import lax
from jax.experimental import pallas as pl
from jax.experimental.pallas import tpu as pltpu
from jax.experimental.pallas import tpu_sc as plsc

_B = 16
_N = 32768
_NC = 2
_NS = 16
_L = 16
_NW = _NC * _NS
_C = _N // _NW
_V = _C // _L

_mesh = plsc.VectorSubcoreMesh(core_axis_name="c", subcore_axis_name="s")


@functools.partial(
    pl.kernel,
    mesh=_mesh,
    out_type=jax.ShapeDtypeStruct((_NC * _B,), jnp.float32),
    scratch_types=[
        pltpu.VMEM((_C,), jnp.float32),
        pltpu.VMEM((_C,), jnp.float32),
        pltpu.VMEM((_C,), jnp.int32),
        pltpu.VMEM((_C,), jnp.float32),
        pltpu.VMEM((_B,), jnp.float32),
        pltpu.VMEM((_NW * _B,), jnp.float32),
        pltpu.VMEM_SHARED((_NW * _B,), jnp.float32),
        pltpu.SemaphoreType.DMA,
    ],
)
def _segsum_sc(scores_hbm, gating_hbm, ids_hbm, out_hbm,
               s_v, g_v, i_v, p_v, part_v, all_v, acc_sh, sem):
    cid = lax.axis_index("c")
    sid = lax.axis_index("s")
    wid = sid * _NC + cid
    base = wid * _C

    c1 = pltpu.async_copy(scores_hbm.at[pl.ds(base, _C)], s_v, sem)
    c2 = pltpu.async_copy(gating_hbm.at[pl.ds(base, _C)], g_v, sem)
    c3 = pltpu.async_copy(ids_hbm.at[pl.ds(base, _C)], i_v, sem)
    c1.wait()
    c2.wait()
    c3.wait()

    def pbody(j, _):
        sl = pl.ds(j * _L, _L)
        p_v[sl] = s_v[sl] * g_v[sl]
        return 0

    lax.fori_loop(0, _V, pbody, 0)

    first = i_v[pl.ds(0, _L)][0]
    last = i_v[pl.ds(_C - _L, _L)][_L - 1]
    lanes = lax.iota(jnp.int32, _L)

    def seg_body(b, part):
        def abody(j, a):
            sl = pl.ds(j * _L, _L)
            return a + jnp.where(i_v[sl] == b, p_v[sl], 0.0)

        acc = lax.fori_loop(0, _V, abody, jnp.zeros((_L,), jnp.float32))
        tot = acc[0]
        for l in range(1, _L):
            tot = tot + acc[l]
        return jnp.where(lanes == b, tot, part)

    part = lax.fori_loop(first, last + 1, seg_body,
                         jnp.zeros((_L,), jnp.float32))
    part_v[...] = part

    pltpu.sync_copy(part_v, acc_sh.at[pl.ds(wid * _B, _B)])
    plsc.subcore_barrier()

    @pl.when(sid == 0)
    def _():
        pltpu.sync_copy(acc_sh, all_v)
        tot = all_v[pl.ds(cid * _B, _B)]
        for t in range(1, _NS):
            tot = tot + all_v[pl.ds((t * _NC + cid) * _B, _B)]
        part_v[...] = tot
        pltpu.sync_copy(part_v, out_hbm.at[pl.ds(cid * _B, _B)])


def kernel(scores, gating, segment_ids):
    partials = _segsum_sc(scores, gating, segment_ids.astype(jnp.int32))
    return partials[:_B] + partials[_B:]
